# baseline (speedup 1.0000x reference)
import functools

import jax
import jax.numpy as jnp
from jax import lax
from jax.experimental import pallas as pl
from jax.experimental.pallas import tpu as pltpu
from jax.experimental.pallas import tpu_sc as plsc

N, D, H, E = 2048, 1024, 2048, 8
T = 256
NTILES = 24
S_PAD = NTILES * T
NW = 32
TPW = N // NW
WL = 128


def _gate_kernel(x_ref, wg_ref, bg_ref, b2_ref, tri_ref, tri8_ref,
                 out0_ref, slot1_ref, slot2_ref, w16a_ref, w16b_ref, te_ref):
    logits = jnp.dot(x_ref[...], wg_ref[...], preferred_element_type=jnp.float32)
    logits = logits + bg_ref[...]
    eidx = jax.lax.broadcasted_iota(jnp.int32, logits.shape, 1)
    i1 = jnp.argmax(logits, axis=-1)
    v1 = jnp.max(logits, axis=-1)
    masked = jnp.where(eidx == i1[:, None], -jnp.inf, logits)
    i2 = jnp.argmax(masked, axis=-1)
    v2 = jnp.max(masked, axis=-1)
    t = jnp.exp(v2 - v1)
    w1 = 1.0 / (1.0 + t)
    w2 = t / (1.0 + t)
    oh1 = eidx == i1[:, None]
    oh2 = eidx == i2[:, None]
    wall = jnp.where(oh1, w1[:, None], jnp.where(oh2, w2[:, None], 0.0))
    out0_ref[...] = jnp.dot(wall, b2_ref[...], preferred_element_type=jnp.float32)

    oh12 = jnp.where(oh1, 1.0, 0.0) + jnp.where(oh2, 1.0, 0.0)
    cumincl = jnp.dot(tri_ref[...], oh12.astype(jnp.bfloat16),
                      preferred_element_type=jnp.float32)
    cumex = cumincl - oh12
    cnt = cumincl[N - 1:N, :]
    ftiles = jnp.floor((cnt + (T - 1.0)) / T)
    pstart_t = jnp.dot(ftiles, tri8_ref[...],
                       preferred_element_type=jnp.float32)
    pstart = pstart_t * T

    slots = pstart + cumex
    slot1 = jnp.sum(jnp.where(oh1, slots, 0.0), axis=1)
    slot2 = jnp.sum(jnp.where(oh2, slots, 0.0), axis=1)
    slot1_ref[...] = slot1[None, :].astype(jnp.int32)
    slot2_ref[...] = slot2[None, :].astype(jnp.int32)

    ones_wl = jnp.ones((1, WL), jnp.float32)
    w16a_ref[...] = w1[:, None] * ones_wl
    w16b_ref[...] = w2[:, None] * ones_wl

    tio = jax.lax.broadcasted_iota(jnp.int32, (E, 32), 1).astype(jnp.float32)
    ge = (tio >= pstart_t.reshape(E, 1)).astype(jnp.float32)
    te = jnp.sum(ge, axis=0) - 1.0
    te_ref[...] = jnp.clip(te[None, :], 0.0, E - 1.0).astype(jnp.int32)


def _gmm_kernel(te_ref, xs_ref, ws_ref, w1_ref, b1_ref, w2_ref, y_ref):
    xb = xs_ref[...].astype(jnp.bfloat16)
    w1b = w1_ref[0].astype(jnp.bfloat16)
    h = jnp.dot(xb, w1b, preferred_element_type=jnp.float32)
    h = jnp.maximum(h + b1_ref[0], 0.0)
    h = (h * ws_ref[:, :1]).astype(jnp.bfloat16)
    w2b = w2_ref[0].astype(jnp.bfloat16)
    y_ref[...] = jnp.dot(h, w2b, preferred_element_type=jnp.float32)


def _dispatch_kernel(x_hbm, s1_hbm, s2_hbm, w16a_hbm, w16b_hbm,
                     xs_hbm, ws_hbm,
                     rows_v, idx1_v, idx2_v, wa_v, wb_v, sem):
    wid = lax.axis_index("s") * 2 + lax.axis_index("c")
    base = wid * TPW
    pltpu.sync_copy(x_hbm.at[pl.ds(base, TPW)], rows_v)
    pltpu.sync_copy(s1_hbm.at[pl.ds(base, TPW)], idx1_v)
    pltpu.sync_copy(s2_hbm.at[pl.ds(base, TPW)], idx2_v)
    pltpu.sync_copy(w16a_hbm.at[pl.ds(base, TPW)], wa_v)
    pltpu.sync_copy(w16b_hbm.at[pl.ds(base, TPW)], wb_v)
    pltpu.async_copy(rows_v, xs_hbm.at[idx1_v], sem).wait()
    pltpu.async_copy(rows_v, xs_hbm.at[idx2_v], sem).wait()
    pltpu.async_copy(wa_v, ws_hbm.at[idx1_v], sem).wait()
    pltpu.async_copy(wb_v, ws_hbm.at[idx2_v], sem).wait()


def _combine_kernel(y_hbm, s1_hbm, s2_hbm, y1_hbm, y2_hbm,
                    g_v, idx1_v, idx2_v, sem):
    wid = lax.axis_index("s") * 2 + lax.axis_index("c")
    base = wid * TPW
    pltpu.sync_copy(s1_hbm.at[pl.ds(base, TPW)], idx1_v)
    pltpu.sync_copy(s2_hbm.at[pl.ds(base, TPW)], idx2_v)
    pltpu.async_copy(y_hbm.at[idx1_v], g_v, sem).wait()
    pltpu.sync_copy(g_v, y1_hbm.at[pl.ds(base, TPW)])
    pltpu.async_copy(y_hbm.at[idx2_v], g_v, sem).wait()
    pltpu.sync_copy(g_v, y2_hbm.at[pl.ds(base, TPW)])


def _sum3_kernel(a_ref, b_ref, c_ref, o_ref):
    o_ref[...] = a_ref[...] + b_ref[...] + c_ref[...]


def kernel(x, Wg, bg, W1, b1, W2, b2):
    tri = jnp.tril(jnp.ones((N, N), jnp.bfloat16))
    tri8 = (jax.lax.broadcasted_iota(jnp.int32, (E, E), 0)
            < jax.lax.broadcasted_iota(jnp.int32, (E, E), 1)).astype(jnp.float32)

    out0, slot1, slot2, w16a, w16b, te = pl.pallas_call(
        _gate_kernel,
        out_shape=(
            jax.ShapeDtypeStruct((N, D), jnp.float32),
            jax.ShapeDtypeStruct((1, N), jnp.int32),
            jax.ShapeDtypeStruct((1, N), jnp.int32),
            jax.ShapeDtypeStruct((N, WL), jnp.float32),
            jax.ShapeDtypeStruct((N, WL), jnp.float32),
            jax.ShapeDtypeStruct((1, 32), jnp.int32),
        ),
    )(x, Wg, bg.reshape(1, E), b2, tri, tri8)

    s1 = slot1.reshape(N)
    s2 = slot2.reshape(N)

    mesh = plsc.VectorSubcoreMesh(core_axis_name="c", subcore_axis_name="s")
    xs, ws = functools.partial(
        pl.kernel,
        mesh=mesh,
        out_type=(
            jax.ShapeDtypeStruct((S_PAD, D), jnp.float32),
            jax.ShapeDtypeStruct((S_PAD, WL), jnp.float32),
        ),
        scratch_types=[
            pltpu.VMEM((TPW, D), jnp.float32),
            pltpu.VMEM((TPW,), jnp.int32),
            pltpu.VMEM((TPW,), jnp.int32),
            pltpu.VMEM((TPW, WL), jnp.float32),
            pltpu.VMEM((TPW, WL), jnp.float32),
            pltpu.SemaphoreType.DMA,
        ],
    )(_dispatch_kernel)(x, s1, s2, w16a, w16b)

    y = pl.pallas_call(
        _gmm_kernel,
        grid_spec=pltpu.PrefetchScalarGridSpec(
            num_scalar_prefetch=1,
            grid=(NTILES,),
            in_specs=[
                pl.BlockSpec((T, D), lambda t, te: (t, 0)),
                pl.BlockSpec((T, WL), lambda t, te: (t, 0)),
                pl.BlockSpec((1, D, H), lambda t, te: (te[t], 0, 0)),
                pl.BlockSpec((1, 1, H), lambda t, te: (te[t], 0, 0)),
                pl.BlockSpec((1, H, D), lambda t, te: (te[t], 0, 0)),
            ],
            out_specs=pl.BlockSpec((T, D), lambda t, te: (t, 0)),
        ),
        out_shape=jax.ShapeDtypeStruct((S_PAD, D), jnp.float32),
    )(te.reshape(32), xs, ws, W1, b1.reshape(E, 1, H), W2)

    y1, y2 = functools.partial(
        pl.kernel,
        mesh=mesh,
        out_type=(
            jax.ShapeDtypeStruct((N, D), jnp.float32),
            jax.ShapeDtypeStruct((N, D), jnp.float32),
        ),
        scratch_types=[
            pltpu.VMEM((TPW, D), jnp.float32),
            pltpu.VMEM((TPW,), jnp.int32),
            pltpu.VMEM((TPW,), jnp.int32),
            pltpu.SemaphoreType.DMA,
        ],
    )(_combine_kernel)(y, s1, s2)

    out = pl.pallas_call(
        _sum3_kernel,
        grid=(4,),
        out_shape=jax.ShapeDtypeStruct((N, D), jnp.float32),
        in_specs=[
            pl.BlockSpec((N // 4, D), lambda i: (i, 0)),
            pl.BlockSpec((N // 4, D), lambda i: (i, 0)),
            pl.BlockSpec((N // 4, D), lambda i: (i, 0)),
        ],
        out_specs=pl.BlockSpec((N // 4, D), lambda i: (i, 0)),
    )(out0, y1, y2)

    return (out, jnp.float32(0.0))


# speedup vs baseline: 1.0325x; 1.0325x over previous
import functools

import jax
import jax.numpy as jnp
from jax import lax
from jax.experimental import pallas as pl
from jax.experimental.pallas import tpu as pltpu
from jax.experimental.pallas import tpu_sc as plsc

N, D, H, E = 2048, 1024, 2048, 8
T = 256
NTILES = 24
S_PAD = NTILES * T
NW = 32
TPW = N // NW
WL = 128


def _gate_kernel(x_ref, wg_ref, bg_ref, b2_ref, tri_ref, tri8_ref,
                 out0_ref, slot1_ref, slot2_ref, w16a_ref, w16b_ref, te_ref,
                 va_ref):
    logits = jnp.dot(x_ref[...], wg_ref[...], preferred_element_type=jnp.float32)
    logits = logits + bg_ref[...]
    eidx = jax.lax.broadcasted_iota(jnp.int32, logits.shape, 1)
    i1 = jnp.argmax(logits, axis=-1)
    v1 = jnp.max(logits, axis=-1)
    masked = jnp.where(eidx == i1[:, None], -jnp.inf, logits)
    i2 = jnp.argmax(masked, axis=-1)
    v2 = jnp.max(masked, axis=-1)
    t = jnp.exp(v2 - v1)
    w1 = 1.0 / (1.0 + t)
    w2 = t / (1.0 + t)
    oh1 = eidx == i1[:, None]
    oh2 = eidx == i2[:, None]
    wall = jnp.where(oh1, w1[:, None], jnp.where(oh2, w2[:, None], 0.0))
    out0_ref[...] = jnp.dot(wall, b2_ref[...], preferred_element_type=jnp.float32)

    oh12 = jnp.where(oh1, 1.0, 0.0) + jnp.where(oh2, 1.0, 0.0)
    cumincl = jnp.dot(tri_ref[...], oh12.astype(jnp.bfloat16),
                      preferred_element_type=jnp.float32)
    cumex = cumincl - oh12
    cnt = cumincl[N - 1:N, :]
    ftiles = jnp.floor((cnt + (T - 1.0)) / T)
    pstart_t = jnp.dot(ftiles, tri8_ref[...],
                       preferred_element_type=jnp.float32)
    pstart = pstart_t * T

    slots = pstart + cumex
    slot1 = jnp.sum(jnp.where(oh1, slots, 0.0), axis=1)
    slot2 = jnp.sum(jnp.where(oh2, slots, 0.0), axis=1)
    slot1_ref[...] = slot1[None, :].astype(jnp.int32)
    slot2_ref[...] = slot2[None, :].astype(jnp.int32)

    ones_wl = jnp.ones((1, WL), jnp.float32)
    w16a_ref[...] = w1[:, None] * ones_wl
    w16b_ref[...] = w2[:, None] * ones_wl

    tio = jax.lax.broadcasted_iota(jnp.int32, (E, 32), 1).astype(jnp.float32)
    ge = (tio >= pstart_t.reshape(E, 1)).astype(jnp.float32)
    te = jnp.sum(ge, axis=0) - 1.0
    te_ref[...] = jnp.clip(te[None, :], 0.0, E - 1.0).astype(jnp.int32)
    total = jnp.sum(ftiles)
    tio1 = jax.lax.broadcasted_iota(jnp.int32, (1, 32), 1).astype(jnp.float32)
    va_ref[...] = (tio1 < total).astype(jnp.int32)


def _gmm_kernel(te_ref, va_ref, xs_ref, ws_ref, w1_ref, b1_ref, w2_ref, y_ref):
    t = pl.program_id(0)

    @pl.when(va_ref[t] == 1)
    def _():
        xb = xs_ref[...].astype(jnp.bfloat16)
        w1b = w1_ref[0].astype(jnp.bfloat16)
        h = jnp.dot(xb, w1b, preferred_element_type=jnp.float32)
        h = jnp.maximum(h + b1_ref[0], 0.0)
        h = (h * ws_ref[:, :1]).astype(jnp.bfloat16)
        w2b = w2_ref[0].astype(jnp.bfloat16)
        y_ref[...] = jnp.dot(h, w2b, preferred_element_type=jnp.float32)


def _dispatch_kernel(x_hbm, s1_hbm, s2_hbm, w16a_hbm, w16b_hbm,
                     xs_hbm, ws_hbm,
                     rows_v, idx1_v, idx2_v, wa_v, wb_v, sem):
    wid = lax.axis_index("s") * 2 + lax.axis_index("c")
    base = wid * TPW
    pltpu.sync_copy(x_hbm.at[pl.ds(base, TPW)], rows_v)
    pltpu.sync_copy(s1_hbm.at[pl.ds(base, TPW)], idx1_v)
    pltpu.sync_copy(s2_hbm.at[pl.ds(base, TPW)], idx2_v)
    pltpu.sync_copy(w16a_hbm.at[pl.ds(base, TPW)], wa_v)
    pltpu.sync_copy(w16b_hbm.at[pl.ds(base, TPW)], wb_v)
    pltpu.async_copy(rows_v, xs_hbm.at[idx1_v], sem).wait()
    pltpu.async_copy(rows_v, xs_hbm.at[idx2_v], sem).wait()
    pltpu.async_copy(wa_v, ws_hbm.at[idx1_v], sem).wait()
    pltpu.async_copy(wb_v, ws_hbm.at[idx2_v], sem).wait()


def _combine_kernel(y_hbm, s1_hbm, s2_hbm, y1_hbm, y2_hbm,
                    g_v, idx1_v, idx2_v, sem):
    wid = lax.axis_index("s") * 2 + lax.axis_index("c")
    base = wid * TPW
    pltpu.sync_copy(s1_hbm.at[pl.ds(base, TPW)], idx1_v)
    pltpu.sync_copy(s2_hbm.at[pl.ds(base, TPW)], idx2_v)
    pltpu.async_copy(y_hbm.at[idx1_v], g_v, sem).wait()
    pltpu.sync_copy(g_v, y1_hbm.at[pl.ds(base, TPW)])
    pltpu.async_copy(y_hbm.at[idx2_v], g_v, sem).wait()
    pltpu.sync_copy(g_v, y2_hbm.at[pl.ds(base, TPW)])


def _sum3_kernel(a_ref, b_ref, c_ref, o_ref):
    o_ref[...] = a_ref[...] + b_ref[...] + c_ref[...]


def kernel(x, Wg, bg, W1, b1, W2, b2):
    tri = jnp.tril(jnp.ones((N, N), jnp.bfloat16))
    tri8 = (jax.lax.broadcasted_iota(jnp.int32, (E, E), 0)
            < jax.lax.broadcasted_iota(jnp.int32, (E, E), 1)).astype(jnp.float32)

    out0, slot1, slot2, w16a, w16b, te, va = pl.pallas_call(
        _gate_kernel,
        out_shape=(
            jax.ShapeDtypeStruct((N, D), jnp.float32),
            jax.ShapeDtypeStruct((1, N), jnp.int32),
            jax.ShapeDtypeStruct((1, N), jnp.int32),
            jax.ShapeDtypeStruct((N, WL), jnp.float32),
            jax.ShapeDtypeStruct((N, WL), jnp.float32),
            jax.ShapeDtypeStruct((1, 32), jnp.int32),
            jax.ShapeDtypeStruct((1, 32), jnp.int32),
        ),
    )(x, Wg, bg.reshape(1, E), b2, tri, tri8)

    s1 = slot1.reshape(N)
    s2 = slot2.reshape(N)

    mesh = plsc.VectorSubcoreMesh(core_axis_name="c", subcore_axis_name="s")
    xs, ws = functools.partial(
        pl.kernel,
        mesh=mesh,
        out_type=(
            jax.ShapeDtypeStruct((S_PAD, D), jnp.float32),
            jax.ShapeDtypeStruct((S_PAD, WL), jnp.float32),
        ),
        scratch_types=[
            pltpu.VMEM((TPW, D), jnp.float32),
            pltpu.VMEM((TPW,), jnp.int32),
            pltpu.VMEM((TPW,), jnp.int32),
            pltpu.VMEM((TPW, WL), jnp.float32),
            pltpu.VMEM((TPW, WL), jnp.float32),
            pltpu.SemaphoreType.DMA,
        ],
    )(_dispatch_kernel)(x, s1, s2, w16a, w16b)

    y = pl.pallas_call(
        _gmm_kernel,
        grid_spec=pltpu.PrefetchScalarGridSpec(
            num_scalar_prefetch=2,
            grid=(NTILES,),
            in_specs=[
                pl.BlockSpec((T, D), lambda t, te, va: (t, 0)),
                pl.BlockSpec((T, WL), lambda t, te, va: (t, 0)),
                pl.BlockSpec((1, D, H), lambda t, te, va: (te[t], 0, 0)),
                pl.BlockSpec((1, 1, H), lambda t, te, va: (te[t], 0, 0)),
                pl.BlockSpec((1, H, D), lambda t, te, va: (te[t], 0, 0)),
            ],
            out_specs=pl.BlockSpec((T, D), lambda t, te, va: (t, 0)),
        ),
        out_shape=jax.ShapeDtypeStruct((S_PAD, D), jnp.float32),
    )(te.reshape(32), va.reshape(32), xs, ws, W1, b1.reshape(E, 1, H), W2)

    y1, y2 = functools.partial(
        pl.kernel,
        mesh=mesh,
        out_type=(
            jax.ShapeDtypeStruct((N, D), jnp.float32),
            jax.ShapeDtypeStruct((N, D), jnp.float32),
        ),
        scratch_types=[
            pltpu.VMEM((TPW, D), jnp.float32),
            pltpu.VMEM((TPW,), jnp.int32),
            pltpu.VMEM((TPW,), jnp.int32),
            pltpu.SemaphoreType.DMA,
        ],
    )(_combine_kernel)(y, s1, s2)

    out = pl.pallas_call(
        _sum3_kernel,
        grid=(4,),
        out_shape=jax.ShapeDtypeStruct((N, D), jnp.float32),
        in_specs=[
            pl.BlockSpec((N // 4, D), lambda i: (i, 0)),
            pl.BlockSpec((N // 4, D), lambda i: (i, 0)),
            pl.BlockSpec((N // 4, D), lambda i: (i, 0)),
        ],
        out_specs=pl.BlockSpec((N // 4, D), lambda i: (i, 0)),
    )(out0, y1, y2)

    return (out, jnp.float32(0.0))


# speedup vs baseline: 1.0497x; 1.0167x over previous
import functools

import jax
import jax.numpy as jnp
from jax import lax
from jax.experimental import pallas as pl
from jax.experimental.pallas import tpu as pltpu
from jax.experimental.pallas import tpu_sc as plsc

N, D, H, E = 2048, 1024, 2048, 8
T = 256
NTILES = 24
S_PAD = NTILES * T
NW = 32
TPW = N // NW
WL = 128


def _gate_kernel(x_ref, wg_ref, bg_ref, tri_ref, tri8_ref,
                 wall_ref, slot1_ref, slot2_ref, w16a_ref, w16b_ref, te_ref,
                 va_ref):
    logits = jnp.dot(x_ref[...], wg_ref[...], preferred_element_type=jnp.float32)
    logits = logits + bg_ref[...]
    eidx = jax.lax.broadcasted_iota(jnp.int32, logits.shape, 1)
    i1 = jnp.argmax(logits, axis=-1)
    v1 = jnp.max(logits, axis=-1)
    masked = jnp.where(eidx == i1[:, None], -jnp.inf, logits)
    i2 = jnp.argmax(masked, axis=-1)
    v2 = jnp.max(masked, axis=-1)
    t = jnp.exp(v2 - v1)
    w1 = 1.0 / (1.0 + t)
    w2 = t / (1.0 + t)
    oh1 = eidx == i1[:, None]
    oh2 = eidx == i2[:, None]
    wall_ref[...] = jnp.where(oh1, w1[:, None], jnp.where(oh2, w2[:, None], 0.0))

    oh12 = jnp.where(oh1, 1.0, 0.0) + jnp.where(oh2, 1.0, 0.0)
    cumincl = jnp.dot(tri_ref[...], oh12.astype(jnp.bfloat16),
                      preferred_element_type=jnp.float32)
    cumex = cumincl - oh12
    cnt = cumincl[N - 1:N, :]
    ftiles = jnp.floor((cnt + (T - 1.0)) / T)
    pstart_t = jnp.dot(ftiles, tri8_ref[...],
                       preferred_element_type=jnp.float32)
    pstart = pstart_t * T

    slots = pstart + cumex
    slot1 = jnp.sum(jnp.where(oh1, slots, 0.0), axis=1)
    slot2 = jnp.sum(jnp.where(oh2, slots, 0.0), axis=1)
    slot1_ref[...] = slot1[None, :].astype(jnp.int32)
    slot2_ref[...] = slot2[None, :].astype(jnp.int32)

    ones_wl = jnp.ones((1, WL), jnp.float32)
    w16a_ref[...] = w1[:, None] * ones_wl
    w16b_ref[...] = w2[:, None] * ones_wl

    tio = jax.lax.broadcasted_iota(jnp.int32, (E, 32), 1).astype(jnp.float32)
    ge = (tio >= pstart_t.reshape(E, 1)).astype(jnp.float32)
    te = jnp.sum(ge, axis=0) - 1.0
    te_ref[...] = jnp.clip(te[None, :], 0.0, E - 1.0).astype(jnp.int32)
    total = jnp.sum(ftiles)
    tio1 = jax.lax.broadcasted_iota(jnp.int32, (1, 32), 1).astype(jnp.float32)
    va_ref[...] = (tio1 < total).astype(jnp.int32)


def _gmm_kernel(te_ref, va_ref, xs_ref, ws_ref, w1_ref, b1_ref, w2_ref, y_ref):
    t = pl.program_id(0)

    @pl.when(va_ref[t] == 1)
    def _():
        xb = xs_ref[...].astype(jnp.bfloat16)
        w1b = w1_ref[0].astype(jnp.bfloat16)
        h = jnp.dot(xb, w1b, preferred_element_type=jnp.float32)
        h = jnp.maximum(h + b1_ref[0], 0.0)
        h = (h * ws_ref[:, :1]).astype(jnp.bfloat16)
        w2b = w2_ref[0].astype(jnp.bfloat16)
        y_ref[...] = jnp.dot(h, w2b, preferred_element_type=jnp.float32)


def _dispatch_kernel(x_hbm, s1_hbm, s2_hbm, w16a_hbm, w16b_hbm,
                     xs_hbm, ws_hbm,
                     rows_v, idx1_v, idx2_v, wa_v, wb_v, sem):
    wid = lax.axis_index("s") * 2 + lax.axis_index("c")
    base = wid * TPW
    pltpu.sync_copy(x_hbm.at[pl.ds(base, TPW)], rows_v)
    pltpu.sync_copy(s1_hbm.at[pl.ds(base, TPW)], idx1_v)
    pltpu.sync_copy(s2_hbm.at[pl.ds(base, TPW)], idx2_v)
    pltpu.sync_copy(w16a_hbm.at[pl.ds(base, TPW)], wa_v)
    pltpu.sync_copy(w16b_hbm.at[pl.ds(base, TPW)], wb_v)
    pltpu.async_copy(rows_v, xs_hbm.at[idx1_v], sem).wait()
    pltpu.async_copy(rows_v, xs_hbm.at[idx2_v], sem).wait()
    pltpu.async_copy(wa_v, ws_hbm.at[idx1_v], sem).wait()
    pltpu.async_copy(wb_v, ws_hbm.at[idx2_v], sem).wait()


def _combine_kernel(y_hbm, s1_hbm, s2_hbm, y1_hbm, y2_hbm,
                    g_v, idx1_v, idx2_v, sem):
    wid = lax.axis_index("s") * 2 + lax.axis_index("c")
    base = wid * TPW
    pltpu.sync_copy(s1_hbm.at[pl.ds(base, TPW)], idx1_v)
    pltpu.sync_copy(s2_hbm.at[pl.ds(base, TPW)], idx2_v)
    pltpu.async_copy(y_hbm.at[idx1_v], g_v, sem).wait()
    pltpu.sync_copy(g_v, y1_hbm.at[pl.ds(base, TPW)])
    pltpu.async_copy(y_hbm.at[idx2_v], g_v, sem).wait()
    pltpu.sync_copy(g_v, y2_hbm.at[pl.ds(base, TPW)])


def _sum3_kernel(wall_ref, b2_ref, b_ref, c_ref, o_ref):
    o_ref[...] = (jnp.dot(wall_ref[...], b2_ref[...],
                          preferred_element_type=jnp.float32)
                  + b_ref[...] + c_ref[...])


def kernel(x, Wg, bg, W1, b1, W2, b2):
    tri = jnp.tril(jnp.ones((N, N), jnp.bfloat16))
    tri8 = (jax.lax.broadcasted_iota(jnp.int32, (E, E), 0)
            < jax.lax.broadcasted_iota(jnp.int32, (E, E), 1)).astype(jnp.float32)

    wall, slot1, slot2, w16a, w16b, te, va = pl.pallas_call(
        _gate_kernel,
        out_shape=(
            jax.ShapeDtypeStruct((N, E), jnp.float32),
            jax.ShapeDtypeStruct((1, N), jnp.int32),
            jax.ShapeDtypeStruct((1, N), jnp.int32),
            jax.ShapeDtypeStruct((N, WL), jnp.float32),
            jax.ShapeDtypeStruct((N, WL), jnp.float32),
            jax.ShapeDtypeStruct((1, 32), jnp.int32),
            jax.ShapeDtypeStruct((1, 32), jnp.int32),
        ),
    )(x, Wg, bg.reshape(1, E), tri, tri8)

    s1 = slot1.reshape(N)
    s2 = slot2.reshape(N)

    mesh = plsc.VectorSubcoreMesh(core_axis_name="c", subcore_axis_name="s")
    xs, ws = functools.partial(
        pl.kernel,
        mesh=mesh,
        out_type=(
            jax.ShapeDtypeStruct((S_PAD, D), jnp.float32),
            jax.ShapeDtypeStruct((S_PAD, WL), jnp.float32),
        ),
        scratch_types=[
            pltpu.VMEM((TPW, D), jnp.float32),
            pltpu.VMEM((TPW,), jnp.int32),
            pltpu.VMEM((TPW,), jnp.int32),
            pltpu.VMEM((TPW, WL), jnp.float32),
            pltpu.VMEM((TPW, WL), jnp.float32),
            pltpu.SemaphoreType.DMA,
        ],
    )(_dispatch_kernel)(x, s1, s2, w16a, w16b)

    y = pl.pallas_call(
        _gmm_kernel,
        grid_spec=pltpu.PrefetchScalarGridSpec(
            num_scalar_prefetch=2,
            grid=(NTILES,),
            in_specs=[
                pl.BlockSpec((T, D), lambda t, te, va: (t, 0)),
                pl.BlockSpec((T, WL), lambda t, te, va: (t, 0)),
                pl.BlockSpec((1, D, H), lambda t, te, va: (te[t], 0, 0)),
                pl.BlockSpec((1, 1, H), lambda t, te, va: (te[t], 0, 0)),
                pl.BlockSpec((1, H, D), lambda t, te, va: (te[t], 0, 0)),
            ],
            out_specs=pl.BlockSpec((T, D), lambda t, te, va: (t, 0)),
        ),
        out_shape=jax.ShapeDtypeStruct((S_PAD, D), jnp.float32),
    )(te.reshape(32), va.reshape(32), xs, ws, W1, b1.reshape(E, 1, H), W2)

    y1, y2 = functools.partial(
        pl.kernel,
        mesh=mesh,
        out_type=(
            jax.ShapeDtypeStruct((N, D), jnp.float32),
            jax.ShapeDtypeStruct((N, D), jnp.float32),
        ),
        scratch_types=[
            pltpu.VMEM((TPW, D), jnp.float32),
            pltpu.VMEM((TPW,), jnp.int32),
            pltpu.VMEM((TPW,), jnp.int32),
            pltpu.SemaphoreType.DMA,
        ],
    )(_combine_kernel)(y, s1, s2)

    out = pl.pallas_call(
        _sum3_kernel,
        grid=(4,),
        out_shape=jax.ShapeDtypeStruct((N, D), jnp.float32),
        in_specs=[
            pl.BlockSpec((N // 4, E), lambda i: (i, 0)),
            pl.BlockSpec((E, D), lambda i: (0, 0)),
            pl.BlockSpec((N // 4, D), lambda i: (i, 0)),
            pl.BlockSpec((N // 4, D), lambda i: (i, 0)),
        ],
        out_specs=pl.BlockSpec((N // 4, D), lambda i: (i, 0)),
    )(wall, b2, y1, y2)

    return (out, jnp.float32(0.0))
